# Initial kernel scaffold; baseline (speedup 1.0000x reference)
import jax
import jax.numpy as jnp
from jax.experimental import pallas as pl


def kernel(x, edge_index, batch, W1, b1, W2, b2, Wfc, bfc):
    raise NotImplementedError("write your pallas kernel here")


# speedup vs baseline: 12.2514x; 12.2514x over previous
import functools

import jax
import jax.numpy as jnp
from jax import lax
from jax.experimental import pallas as pl
from jax.experimental.pallas import tpu as pltpu
from jax.experimental.pallas import tpu_sc as plsc

N = 10000
E = 320000
D = 128

NC = 2
NS = 16
NW = NC * NS
EPT = E // NW
K = 80
NCHUNK = EPT // K
NP = 10240
RPS = NP // NS
ZR = 128
DW = 16

_mesh = plsc.VectorSubcoreMesh(core_axis_name="c", subcore_axis_name="s")


def _deg_body(dstv, ones_h, zeros_h, out, dstidx, onesv, zbuf, acc):
    c = lax.axis_index("c")
    s = lax.axis_index("s")
    wid = c * NS + s
    pltpu.sync_copy(zeros_h, zbuf)
    for j in range(RPS // ZR):
        pltpu.sync_copy(zbuf, acc.at[pl.ds(s * RPS + j * ZR, ZR)])
    pltpu.sync_copy(ones_h, onesv)
    plsc.subcore_barrier()
    base = wid * EPT

    def step(i, carry):
        off = pl.multiple_of(base + i * K, 8)
        pltpu.sync_copy(dstv.at[pl.ds(off, K)], dstidx)
        pltpu.sync_copy(onesv, acc.at[dstidx], add=True)
        return carry

    lax.fori_loop(0, NCHUNK, step, 0)
    plsc.subcore_barrier()
    for j in range(RPS // ZR):
        r0 = s * RPS + j * ZR
        pltpu.sync_copy(acc.at[pl.ds(r0, ZR)], zbuf)
        pltpu.sync_copy(zbuf, out.at[pl.ds(c * NP + r0, ZR)])


_deg = functools.partial(
    pl.kernel,
    out_type=jax.ShapeDtypeStruct((NC * NP, D), jnp.float32),
    mesh=_mesh,
    scratch_types=[
        pltpu.VMEM((K,), jnp.int32),
        pltpu.VMEM((K, D), jnp.float32),
        pltpu.VMEM((ZR, D), jnp.float32),
        pltpu.VMEM_SHARED((NP, D), jnp.float32),
    ],
)(_deg_body)


def _scatter_body(hs, srcv, dstv, zeros_h, out, srcidx, dstidx, rows, zbuf, acc,
                  sem):
    c = lax.axis_index("c")
    s = lax.axis_index("s")
    wid = c * NS + s
    pltpu.sync_copy(zeros_h, zbuf)
    for j in range(RPS // ZR):
        pltpu.sync_copy(zbuf, acc.at[pl.ds(s * RPS + j * ZR, ZR)])
    plsc.subcore_barrier()
    base = wid * EPT

    def step(i, carry):
        off = pl.multiple_of(base + i * K, 8)
        pltpu.sync_copy(srcv.at[pl.ds(off, K)], srcidx)
        pltpu.sync_copy(dstv.at[pl.ds(off, K)], dstidx)
        pltpu.async_copy(hs.at[srcidx], rows, sem).wait()
        pltpu.sync_copy(rows, acc.at[dstidx], add=True)
        return carry

    lax.fori_loop(0, NCHUNK, step, 0)
    plsc.subcore_barrier()
    for j in range(RPS // ZR):
        r0 = s * RPS + j * ZR
        pltpu.sync_copy(acc.at[pl.ds(r0, ZR)], zbuf)
        pltpu.sync_copy(zbuf, out.at[pl.ds(c * NP + r0, ZR)])


_scatter = functools.partial(
    pl.kernel,
    out_type=jax.ShapeDtypeStruct((NC * NP, D), jnp.float32),
    mesh=_mesh,
    scratch_types=[
        pltpu.VMEM((K,), jnp.int32),
        pltpu.VMEM((K,), jnp.int32),
        pltpu.VMEM((K, D), jnp.float32),
        pltpu.VMEM((ZR, D), jnp.float32),
        pltpu.VMEM_SHARED((NP, D), jnp.float32),
        pltpu.SemaphoreType.DMA,
    ],
)(_scatter_body)


BR = 1000


def _tc1_body(dp_ref, x_ref, w_ref, hs_ref, dinv_ref):
    deg = 1.0 + dp_ref[0][:, :1] + dp_ref[1][:, :1]
    dinv = lax.rsqrt(deg)
    h = jnp.dot(x_ref[...], w_ref[...], preferred_element_type=jnp.float32)
    hs_ref[...] = h * dinv
    dinv_ref[...] = jnp.broadcast_to(dinv, (BR, DW))


def _tc1(degp, x, w):
    return pl.pallas_call(
        _tc1_body,
        grid=(N // BR,),
        in_specs=[
            pl.BlockSpec((NC, BR, D), lambda i: (0, i, 0)),
            pl.BlockSpec((BR, D), lambda i: (i, 0)),
            pl.BlockSpec((D, D), lambda i: (0, 0)),
        ],
        out_specs=[
            pl.BlockSpec((BR, D), lambda i: (i, 0)),
            pl.BlockSpec((BR, DW), lambda i: (i, 0)),
        ],
        out_shape=[
            jax.ShapeDtypeStruct((N, D), jnp.float32),
            jax.ShapeDtypeStruct((N, DW), jnp.float32),
        ],
    )(degp, x, w)


def _tc2_body(pp_ref, hs_ref, dinv_ref, b_ref, w_ref, out_ref):
    dinv = dinv_ref[...][:, :1]
    a = (pp_ref[0] + pp_ref[1] + hs_ref[...]) * dinv + b_ref[...]
    h = jnp.maximum(a, 0.0)
    out_ref[...] = jnp.dot(
        h, w_ref[...], preferred_element_type=jnp.float32) * dinv


def _tc2(pp, hs, dinv, b, w):
    return pl.pallas_call(
        _tc2_body,
        grid=(N // BR,),
        in_specs=[
            pl.BlockSpec((NC, BR, D), lambda i: (0, i, 0)),
            pl.BlockSpec((BR, D), lambda i: (i, 0)),
            pl.BlockSpec((BR, DW), lambda i: (i, 0)),
            pl.BlockSpec((1, D), lambda i: (0, 0)),
            pl.BlockSpec((D, D), lambda i: (0, 0)),
        ],
        out_specs=pl.BlockSpec((BR, D), lambda i: (i, 0)),
        out_shape=jax.ShapeDtypeStruct((N, D), jnp.float32),
    )(pp, hs, dinv, b, w)


def _tc3_body(pp_ref, hs_ref, dinv_ref, b_ref, wfc_ref, bfc_ref, out_ref):
    dinv = dinv_ref[...][:, :1]
    h = jnp.maximum(
        (pp_ref[0] + pp_ref[1] + hs_ref[...]) * dinv + b_ref[...], 0.0)
    out_ref[...] = jnp.dot(
        h, wfc_ref[...], preferred_element_type=jnp.float32) + bfc_ref[...]


def _tc3(pp, hs, dinv, b, wfc, bfc):
    return pl.pallas_call(
        _tc3_body,
        grid=(N // BR,),
        in_specs=[
            pl.BlockSpec((NC, BR, D), lambda i: (0, i, 0)),
            pl.BlockSpec((BR, D), lambda i: (i, 0)),
            pl.BlockSpec((BR, DW), lambda i: (i, 0)),
            pl.BlockSpec((1, D), lambda i: (0, 0)),
            pl.BlockSpec((D, D), lambda i: (0, 0)),
            pl.BlockSpec((1, D), lambda i: (0, 0)),
        ],
        out_specs=pl.BlockSpec((BR, D), lambda i: (i, 0)),
        out_shape=jax.ShapeDtypeStruct((N, D), jnp.float32),
    )(pp, hs, dinv, b, wfc, bfc)


def kernel(x, edge_index, batch, W1, b1, W2, b2, Wfc, bfc):
    src = edge_index[0].astype(jnp.int32)
    dst = edge_index[1].astype(jnp.int32)
    zeros_rows = jnp.zeros((ZR, D), jnp.float32)
    ones_k = jnp.ones((K, D), jnp.float32)

    degp = _deg(dst, ones_k, zeros_rows).reshape(NC, NP, D)
    hs1, dinv = _tc1(degp, x, W1)
    p1 = _scatter(hs1, src, dst, zeros_rows).reshape(NC, NP, D)
    hs2 = _tc2(p1, hs1, dinv, b1.reshape(1, D), W2)
    p2 = _scatter(hs2, src, dst, zeros_rows).reshape(NC, NP, D)
    out = _tc3(p2, hs2, dinv, b2.reshape(1, D), Wfc, bfc.reshape(1, D))
    return out
